# baseline (speedup 1.0000x reference)
import dataclasses

import jax
import jax.numpy as jnp
from jax.experimental import pallas as pl
from jax.experimental.pallas import tpu as pltpu
from jax.experimental.pallas import tpu_sc as plsc

E = 16
H = 1024
FFN = 2048
TOP_K = 2
T = 128
DEPTH = 3
ROWS = 4

_VEC_MESH = plsc.VectorSubcoreMesh(core_axis_name="core",
                                   subcore_axis_name="subcore")


def _gx_body(x_ref, wr_ref, gx_ref):
    gx_ref[...] = jax.lax.dot_general(
        x_ref[...], wr_ref[...],
        dimension_numbers=(((1,), (1,)), ((), ())),
        preferred_element_type=jnp.float32)


def _router_probs_topk(gx):
    cp = pltpu.CompilerParams()
    if "needs_layout_passes" in pltpu.CompilerParams.__dataclass_fields__:
        cp = dataclasses.replace(cp, needs_layout_passes=False)

    @pl.kernel(out_type=[jax.ShapeDtypeStruct((T, E), jnp.float32),
                         jax.ShapeDtypeStruct((T, E), jnp.int32)],
               mesh=_VEC_MESH, compiler_params=cp)
    def k(gx_hbm, probs_hbm, topk_hbm):
        def body(g_vmem, p_vmem, t_vmem):
            @pl.loop(0, ROWS)
            def _(j):
                row = g_vmem.at[j][...]
                lane = jax.lax.iota(jnp.int32, 16)
                m = jnp.max(row)
                ep = jnp.exp(row - m)
                p = ep / jnp.sum(ep)
                p_vmem.at[j][...] = p
                m1 = jnp.max(p)
                i1 = jnp.min(jnp.where(p == m1, lane, E))
                pm = jnp.where(lane == i1, -1.0, p)
                m2 = jnp.max(pm)
                i2 = jnp.min(jnp.where(pm == m2, lane, E))
                t_vmem.at[j][...] = jnp.where(
                    lane == 0, i1, jnp.where(lane == 1, i2, 0))

        pltpu.emit_pipeline(
            body,
            grid=(T // ROWS,),
            in_specs=[pl.BlockSpec((ROWS, E), lambda i: (i, 0))],
            out_specs=[pl.BlockSpec((ROWS, E), lambda i: (i, 0)),
                       pl.BlockSpec((ROWS, E), lambda i: (i, 0))],
            core_axis_name=("core", "subcore"),
            dimension_semantics=(pltpu.PARALLEL,),
        )(gx_hbm, probs_hbm, topk_hbm)

    return k(gx)


def _moe_body(x_ref, wr_ref, w1_hbm, b1_ref, w2_hbm, b2_ref,
              out_ref, w1_buf, w2_buf, c_ref, sem1, sem2):
    def start_copy(e, slot):
        pltpu.make_async_copy(w1_hbm.at[e], w1_buf.at[slot],
                              sem1.at[slot]).start()
        pltpu.make_async_copy(w2_hbm.at[e], w2_buf.at[slot],
                              sem2.at[slot]).start()

    for k in range(DEPTH):
        start_copy(k, k)

    xx = x_ref[...]
    g = jax.lax.dot_general(
        xx, wr_ref[...],
        dimension_numbers=(((1,), (1,)), ((), ())),
        preferred_element_type=jnp.float32)
    m = jnp.max(g, axis=-1, keepdims=True)
    ex = jnp.exp(g - m)
    p = ex / jnp.sum(ex, axis=-1, keepdims=True)
    cols = jax.lax.broadcasted_iota(jnp.int32, (T, E), 1)
    i1 = jnp.argmax(p, axis=-1)
    p1 = jnp.max(p, axis=-1)
    pm = jnp.where(cols == i1[:, None], -1.0, p)
    i2 = jnp.argmax(pm, axis=-1)
    p2 = jnp.max(pm, axis=-1)
    s = p1 + p2 + 1e-9
    p1n = (p1 / s)[:, None]
    p2n = (p2 / s)[:, None]
    c_ref[...] = (jnp.where(cols == i1[:, None], p1n, 0.0)
                  + jnp.where(cols == i2[:, None], p2n, 0.0))
    out_ref[...] = jnp.zeros_like(out_ref)

    def step(e, carry):
        slot = jax.lax.rem(e, DEPTH)
        pltpu.make_async_copy(w1_hbm.at[e], w1_buf.at[slot],
                              sem1.at[slot]).wait()
        pltpu.make_async_copy(w2_hbm.at[e], w2_buf.at[slot],
                              sem2.at[slot]).wait()

        cols_ = jax.lax.broadcasted_iota(jnp.int32, (T, E), 1)
        coef = jnp.sum(jnp.where(cols_ == e, c_ref[...], 0.0), axis=1,
                       keepdims=True)
        h = jax.lax.dot_general(
            x_ref[...], w1_buf[slot],
            dimension_numbers=(((1,), (1,)), ((), ())),
            preferred_element_type=jnp.float32)
        h = h + b1_ref[e]
        h = 0.5 * h * (1.0 + jax.lax.erf(h * 0.7071067811865476))
        o = jax.lax.dot_general(
            h, w2_buf[slot],
            dimension_numbers=(((1,), (1,)), ((), ())),
            preferred_element_type=jnp.float32)
        out_ref[...] += coef * (o + b2_ref[e])

        @pl.when(e + DEPTH < E)
        def _next():
            start_copy(e + DEPTH, slot)

        return carry

    jax.lax.fori_loop(0, E, step, 0)


@jax.jit
def kernel(x, Wr, W1, b1, W2, b2):
    B, S, _ = x.shape
    xf = x.reshape(T, H)

    gx = pl.pallas_call(
        _gx_body,
        in_specs=[pl.BlockSpec(memory_space=pltpu.VMEM),
                  pl.BlockSpec(memory_space=pltpu.VMEM)],
        out_specs=pl.BlockSpec(memory_space=pltpu.VMEM),
        out_shape=jax.ShapeDtypeStruct((T, E), jnp.float32),
    )(xf, Wr)

    probs, topk16 = _router_probs_topk(gx)

    out = pl.pallas_call(
        _moe_body,
        in_specs=[
            pl.BlockSpec(memory_space=pltpu.VMEM),
            pl.BlockSpec(memory_space=pltpu.VMEM),
            pl.BlockSpec(memory_space=pltpu.HBM),
            pl.BlockSpec(memory_space=pltpu.VMEM),
            pl.BlockSpec(memory_space=pltpu.HBM),
            pl.BlockSpec(memory_space=pltpu.VMEM),
        ],
        out_specs=pl.BlockSpec(memory_space=pltpu.VMEM),
        out_shape=jax.ShapeDtypeStruct((T, H), jnp.float32),
        scratch_shapes=[
            pltpu.VMEM((DEPTH, FFN, H), jnp.float32),
            pltpu.VMEM((DEPTH, H, FFN), jnp.float32),
            pltpu.VMEM((T, E), jnp.float32),
            pltpu.SemaphoreType.DMA((DEPTH,)),
            pltpu.SemaphoreType.DMA((DEPTH,)),
        ],
        compiler_params=pltpu.CompilerParams(
            vmem_limit_bytes=100 * 1024 * 1024,
        ),
    )(xf, Wr, W1, b1.reshape(E, 1, FFN), W2, b2.reshape(E, 1, H))

    return (out.reshape(B, S, H), probs.reshape(B, S, E),
            topk16[:, :TOP_K].reshape(B, S, TOP_K))


# speedup vs baseline: 1.1748x; 1.1748x over previous
import jax
import jax.numpy as jnp
from jax.experimental import pallas as pl
from jax.experimental.pallas import tpu as pltpu

E = 16
H = 1024
FFN = 2048
TOP_K = 2
T = 128
DEPTH = 3


def _moe_body(x_ref, wr_ref, w1_hbm, b1_ref, w2_hbm, b2_ref,
              out_ref, probs_ref, topk_ref,
              w1_buf, w2_buf, c_ref, sem1, sem2):
    def start_copy(e, slot):
        pltpu.make_async_copy(w1_hbm.at[e], w1_buf.at[slot],
                              sem1.at[slot]).start()
        pltpu.make_async_copy(w2_hbm.at[e], w2_buf.at[slot],
                              sem2.at[slot]).start()

    for k in range(DEPTH):
        start_copy(k, k)

    xx = x_ref[...]
    g = jax.lax.dot_general(
        xx, wr_ref[...],
        dimension_numbers=(((1,), (1,)), ((), ())),
        preferred_element_type=jnp.float32)
    m = jnp.max(g, axis=-1, keepdims=True)
    ex = jnp.exp(g - m)
    p = ex / jnp.sum(ex, axis=-1, keepdims=True)
    probs_ref[...] = p
    cols = jax.lax.broadcasted_iota(jnp.int32, (T, E), 1)
    i1 = jnp.argmax(p, axis=-1)
    p1 = jnp.max(p, axis=-1)
    pm = jnp.where(cols == i1[:, None], -1.0, p)
    i2 = jnp.argmax(pm, axis=-1)
    p2 = jnp.max(pm, axis=-1)
    s = p1 + p2 + 1e-9
    p1n = (p1 / s)[:, None]
    p2n = (p2 / s)[:, None]
    topk_ref[...] = jnp.concatenate([i1[:, None], i2[:, None]], axis=1)
    c_ref[...] = (jnp.where(cols == i1[:, None], p1n, 0.0)
                  + jnp.where(cols == i2[:, None], p2n, 0.0))
    out_ref[...] = jnp.zeros_like(out_ref)

    def step(e, carry):
        slot = jax.lax.rem(e, DEPTH)
        pltpu.make_async_copy(w1_hbm.at[e], w1_buf.at[slot],
                              sem1.at[slot]).wait()
        pltpu.make_async_copy(w2_hbm.at[e], w2_buf.at[slot],
                              sem2.at[slot]).wait()

        cols_ = jax.lax.broadcasted_iota(jnp.int32, (T, E), 1)
        coef = jnp.sum(jnp.where(cols_ == e, c_ref[...], 0.0), axis=1,
                       keepdims=True)
        h = jax.lax.dot_general(
            x_ref[...], w1_buf[slot],
            dimension_numbers=(((1,), (1,)), ((), ())),
            preferred_element_type=jnp.float32)
        h = h + b1_ref[e]
        h = 0.5 * h * (1.0 + jax.lax.erf(h * 0.7071067811865476))
        o = jax.lax.dot_general(
            h, w2_buf[slot],
            dimension_numbers=(((1,), (1,)), ((), ())),
            preferred_element_type=jnp.float32)
        out_ref[...] += coef * (o + b2_ref[e])

        @pl.when(e + DEPTH < E)
        def _next():
            start_copy(e + DEPTH, slot)

        return carry

    jax.lax.fori_loop(0, E, step, 0)


@jax.jit
def kernel(x, Wr, W1, b1, W2, b2):
    B, S, _ = x.shape
    xf = x.reshape(T, H)

    out, probs, topk = pl.pallas_call(
        _moe_body,
        in_specs=[
            pl.BlockSpec(memory_space=pltpu.VMEM),
            pl.BlockSpec(memory_space=pltpu.VMEM),
            pl.BlockSpec(memory_space=pltpu.HBM),
            pl.BlockSpec(memory_space=pltpu.VMEM),
            pl.BlockSpec(memory_space=pltpu.HBM),
            pl.BlockSpec(memory_space=pltpu.VMEM),
        ],
        out_specs=[
            pl.BlockSpec(memory_space=pltpu.VMEM),
            pl.BlockSpec(memory_space=pltpu.VMEM),
            pl.BlockSpec(memory_space=pltpu.VMEM),
        ],
        out_shape=[
            jax.ShapeDtypeStruct((T, H), jnp.float32),
            jax.ShapeDtypeStruct((T, E), jnp.float32),
            jax.ShapeDtypeStruct((T, TOP_K), jnp.int32),
        ],
        scratch_shapes=[
            pltpu.VMEM((DEPTH, FFN, H), jnp.float32),
            pltpu.VMEM((DEPTH, H, FFN), jnp.float32),
            pltpu.VMEM((T, E), jnp.float32),
            pltpu.SemaphoreType.DMA((DEPTH,)),
            pltpu.SemaphoreType.DMA((DEPTH,)),
        ],
        compiler_params=pltpu.CompilerParams(
            vmem_limit_bytes=100 * 1024 * 1024,
        ),
    )(xf, Wr, W1, b1.reshape(E, 1, FFN), W2, b2.reshape(E, 1, H))

    return (out.reshape(B, S, H), probs.reshape(B, S, E),
            topk.reshape(B, S, TOP_K))
